# baseline (speedup 1.0000x reference)
import functools

import jax
import jax.numpy as jnp
from jax import lax
from jax.experimental import pallas as pl
from jax.experimental.pallas import tpu as pltpu
from jax.experimental.pallas import tpu_sc as plsc

BATCH = 16384
EMBED_DIM = 32
NUM_CORES = 2
NUM_SUBCORES = 16
NUM_WORKERS = NUM_CORES * NUM_SUBCORES
SIDE_WORKERS = NUM_WORKERS // 2
BLK = 128
LANES = 16
GROUP = 16
NSLOT = 8

SC_N = 8192
TC_N = BATCH - SC_N
SC_PER_WORKER = SC_N // SIDE_WORKERS
SC_NGROUPS = SC_PER_WORKER // GROUP
IPB = 8


def _sc_kernel(
    u_idx_hbm, i_idx_hbm,
    ue_hbm, ie_hbm, up_hbm, ip_hbm,
    u_out_hbm, i_out_hbm,
    idx_v, e_blk, p_blk, out_v,
    sem_e, sem_p,
):
    w = lax.axis_index("s") * NUM_CORES + lax.axis_index("c")

    rows_lo = lax.iota(jnp.int32, LANES)
    rows_hi = rows_lo + LANES
    third = jnp.float32(1.0 / 3.0)

    def side_worker(idx_hbm, e_hbm, p_hbm, out_hbm, lw):
        b0 = lw * SC_PER_WORKER
        pltpu.sync_copy(idx_hbm.at[pl.ds(b0, SC_PER_WORKER)], idx_v)

        def fire(u, slot):
            ub = (u >> 7) * BLK
            pltpu.async_copy(e_hbm.at[:, pl.ds(ub, BLK)], e_blk.at[slot], sem_e)
            pltpu.async_copy(p_hbm.at[:, pl.ds(ub, BLK)], p_blk.at[slot], sem_p)

        def drain(slot):
            pltpu.make_async_copy(e_hbm.at[:, pl.ds(0, BLK)], e_blk.at[slot], sem_e).wait()
            pltpu.make_async_copy(p_hbm.at[:, pl.ds(0, BLK)], p_blk.at[slot], sem_p).wait()

        def process(u, slot, col):
            lane_vec = jnp.full((LANES,), u & (BLK - 1), jnp.int32)
            col_v = jnp.full((LANES,), col, jnp.int32)
            e_lo = plsc.load_gather(e_blk.at[slot], [rows_lo, lane_vec])
            e_hi = plsc.load_gather(e_blk.at[slot], [rows_hi, lane_vec])
            p_lo = plsc.load_gather(p_blk.at[slot], [rows_lo, lane_vec])
            p_hi = plsc.load_gather(p_blk.at[slot], [rows_hi, lane_vec])
            o_lo = (e_lo + p_lo + e_lo * p_lo) * third
            o_hi = (e_hi + p_hi + e_hi * p_hi) * third
            plsc.store_scatter(out_v, [rows_lo, col_v], o_lo)
            plsc.store_scatter(out_v, [rows_hi, col_v], o_hi)

        vec0 = idx_v[pl.ds(0, GROUP)]
        for k in range(NSLOT - 1):
            fire(vec0[k], k)

        @pl.loop(0, SC_NGROUPS)
        def _(g):
            vec = idx_v[pl.ds(g * GROUP, GROUP)]
            nxt = idx_v[pl.ds((g + 1) * GROUP % SC_PER_WORKER, GROUP)]
            for k in range(GROUP):
                ka = k + NSLOT - 1
                if ka < GROUP:
                    fire(vec[ka], ka & (NSLOT - 1))
                else:
                    fire(nxt[ka - GROUP], ka & (NSLOT - 1))
                drain(k & (NSLOT - 1))
                process(vec[k], k & (NSLOT - 1), g * GROUP + k)

        for k in range(NSLOT - 1):
            drain(k)

        pltpu.sync_copy(out_v, out_hbm.at[:, pl.ds(b0, SC_PER_WORKER)])

    @pl.when(w < SIDE_WORKERS)
    def _():
        side_worker(u_idx_hbm, ue_hbm, up_hbm, u_out_hbm, w)

    @pl.when(w >= SIDE_WORKERS)
    def _():
        side_worker(i_idx_hbm, ie_hbm, ip_hbm, i_out_hbm, w - SIDE_WORKERS)


def _sc_gather(u_idx, i_idx, ue_t, ie_t, up_t, ip_t):
    mesh = plsc.VectorSubcoreMesh(core_axis_name="c", subcore_axis_name="s")
    out_t = jax.ShapeDtypeStruct((EMBED_DIM, SC_N), jnp.float32)
    blk = pltpu.VMEM((NSLOT, EMBED_DIM, BLK), jnp.float32)

    run = pl.kernel(
        _sc_kernel,
        out_type=(out_t, out_t),
        mesh=mesh,
        compiler_params=pltpu.CompilerParams(needs_layout_passes=False),
        scratch_types=[
            pltpu.VMEM((SC_PER_WORKER,), jnp.int32),
            blk, blk,
            pltpu.VMEM((EMBED_DIM, SC_PER_WORKER), jnp.float32),
            pltpu.SemaphoreType.DMA,
            pltpu.SemaphoreType.DMA,
        ],
    )
    return run(u_idx, i_idx, ue_t, ie_t, up_t, ip_t)


def _tc_kernel(u_sref, i_sref, *refs):
    ue = refs[0:IPB]
    up = refs[IPB:2 * IPB]
    ie = refs[2 * IPB:3 * IPB]
    ip = refs[3 * IPB:4 * IPB]
    uo_ref, io_ref = refs[4 * IPB], refs[4 * IPB + 1]
    pid = pl.program_id(0)
    third = jnp.float32(1.0 / 3.0)
    lane_iota = lax.broadcasted_iota(jnp.int32, (EMBED_DIM, BLK), 1)
    sub = pid % (BLK // IPB)

    def side(sref, e_refs, p_refs, o_ref):
        acc = jnp.zeros((EMBED_DIM, BLK), jnp.float32)
        for j in range(IPB):
            lane = sref[pid * IPB + j] & (BLK - 1)
            e = e_refs[j][...]
            p = p_refs[j][...]
            c = (e + p + e * p) * third
            col = jnp.sum(jnp.where(lane_iota == lane, c, 0.0),
                          axis=1, keepdims=True)
            acc = acc + jnp.where(lane_iota == sub * IPB + j, col, 0.0)
        prev = o_ref[...]
        o_ref[...] = jnp.where(sub == 0, acc, prev + acc)

    side(u_sref, ue, up, uo_ref)
    side(i_sref, ie, ip, io_ref)


def _tc_gather(u_idx, i_idx, ue_t, ie_t, up_t, ip_t):
    tbl_spec = lambda sel: pl.BlockSpec(
        (EMBED_DIM, BLK), index_map=sel)

    def u_sel(j):
        return lambda i, uref, iref: (0, uref[i * IPB + j] >> 7)

    def i_sel(j):
        return lambda i, uref, iref: (0, iref[i * IPB + j] >> 7)

    in_specs = ([tbl_spec(u_sel(j)) for j in range(IPB)]
                + [tbl_spec(u_sel(j)) for j in range(IPB)]
                + [tbl_spec(i_sel(j)) for j in range(IPB)]
                + [tbl_spec(i_sel(j)) for j in range(IPB)])
    out_spec = pl.BlockSpec((EMBED_DIM, BLK),
                            lambda i, uref, iref: (0, i // (BLK // IPB)))
    out_t = jax.ShapeDtypeStruct((EMBED_DIM, TC_N), jnp.float32)

    grid_spec = pltpu.PrefetchScalarGridSpec(
        num_scalar_prefetch=2,
        grid=(TC_N // IPB,),
        in_specs=in_specs,
        out_specs=[out_spec, out_spec],
    )
    run = pl.pallas_call(
        _tc_kernel,
        grid_spec=grid_spec,
        out_shape=[out_t, out_t],
        compiler_params=pltpu.CompilerParams(
            dimension_semantics=("parallel",)),
    )
    args = ([ue_t] * IPB) + ([up_t] * IPB) + ([ie_t] * IPB) + ([ip_t] * IPB)
    return run(u_idx, i_idx, *args)


def kernel(user_indices, item_indices, user_embedding_table,
           item_embedding_table, user_profiles, item_profiles):
    u_idx = user_indices.astype(jnp.int32)
    i_idx = item_indices.astype(jnp.int32)
    ue_t = user_embedding_table.T
    ie_t = item_embedding_table.T
    up_t = user_profiles.T
    ip_t = item_profiles.T

    u_sc, i_sc = _sc_gather(u_idx[:SC_N], i_idx[:SC_N],
                            ue_t, ie_t, up_t, ip_t)
    u_tc, i_tc = _tc_gather(u_idx[SC_N:], i_idx[SC_N:],
                            ue_t, ie_t, up_t, ip_t)

    u_out = jnp.concatenate([u_sc, u_tc], axis=1)
    i_out = jnp.concatenate([i_sc, i_tc], axis=1)
    return (u_out.T, i_out.T)


# speedup vs baseline: 2.3854x; 2.3854x over previous
import jax
import jax.numpy as jnp
from jax import lax
from jax.experimental import pallas as pl
from jax.experimental.pallas import tpu as pltpu
from jax.experimental.pallas import tpu_sc as plsc

BATCH = 16384
EMBED_DIM = 32
NUM_CORES = 2
NUM_SUBCORES = 16
NUM_WORKERS = NUM_CORES * NUM_SUBCORES
SIDE_WORKERS = NUM_WORKERS // 2
PER_WORKER = BATCH // SIDE_WORKERS
BLK = 128
LANES = 16
GROUP = 16
NGROUPS = PER_WORKER // GROUP
NSLOT = 8


def _embed_kernel(
    u_idx_hbm, i_idx_hbm,
    ue_hbm, ie_hbm, up_hbm, ip_hbm,
    u_out_hbm, i_out_hbm,
    idx_v, e_blk, p_blk, out_v,
    sem_e, sem_p,
):
    w = lax.axis_index("s") * NUM_CORES + lax.axis_index("c")

    rows_lo = lax.iota(jnp.int32, LANES)
    rows_hi = rows_lo + LANES
    third = jnp.float32(1.0 / 3.0)

    def side_worker(idx_hbm, e_hbm, p_hbm, out_hbm, lw):
        b0 = lw * PER_WORKER
        pltpu.sync_copy(idx_hbm.at[pl.ds(b0, PER_WORKER)], idx_v)

        def fire(u, slot):
            ub = (u >> 7) * BLK
            for t in range(EMBED_DIM // 8):
                pltpu.async_copy(e_hbm.at[pl.ds(8 * t, 8), pl.ds(ub, BLK)],
                                 e_blk.at[slot].at[pl.ds(8 * t, 8)], sem_e)
                pltpu.async_copy(p_hbm.at[pl.ds(8 * t, 8), pl.ds(ub, BLK)],
                                 p_blk.at[slot].at[pl.ds(8 * t, 8)], sem_p)

        def drain(slot):
            for t in range(EMBED_DIM // 8):
                pltpu.make_async_copy(e_hbm.at[pl.ds(8 * t, 8), pl.ds(0, BLK)],
                                      e_blk.at[slot].at[pl.ds(8 * t, 8)], sem_e).wait()
                pltpu.make_async_copy(p_hbm.at[pl.ds(8 * t, 8), pl.ds(0, BLK)],
                                      p_blk.at[slot].at[pl.ds(8 * t, 8)], sem_p).wait()

        def process(u, slot, col):
            lane_vec = jnp.full((LANES,), u & (BLK - 1), jnp.int32)
            col_v = jnp.full((LANES,), col, jnp.int32)
            e_lo = plsc.load_gather(e_blk.at[slot], [rows_lo, lane_vec])
            e_hi = plsc.load_gather(e_blk.at[slot], [rows_hi, lane_vec])
            p_lo = plsc.load_gather(p_blk.at[slot], [rows_lo, lane_vec])
            p_hi = plsc.load_gather(p_blk.at[slot], [rows_hi, lane_vec])
            o_lo = (e_lo + p_lo + e_lo * p_lo) * third
            o_hi = (e_hi + p_hi + e_hi * p_hi) * third
            plsc.store_scatter(out_v, [rows_lo, col_v], o_lo)
            plsc.store_scatter(out_v, [rows_hi, col_v], o_hi)

        vec0 = idx_v[pl.ds(0, GROUP)]
        for k in range(NSLOT - 1):
            fire(vec0[k], k)

        @pl.loop(0, NGROUPS)
        def _(g):
            vec = idx_v[pl.ds(g * GROUP, GROUP)]
            nxt = idx_v[pl.ds((g + 1) * GROUP % PER_WORKER, GROUP)]
            for k in range(GROUP):
                ka = k + NSLOT - 1
                if ka < GROUP:
                    fire(vec[ka], ka & (NSLOT - 1))
                else:
                    fire(nxt[ka - GROUP], ka & (NSLOT - 1))
                drain(k & (NSLOT - 1))
                process(vec[k], k & (NSLOT - 1), g * GROUP + k)

        for k in range(NSLOT - 1):
            drain(k)

        pltpu.sync_copy(out_v, out_hbm.at[:, pl.ds(b0, PER_WORKER)])

    @pl.when(w < SIDE_WORKERS)
    def _():
        side_worker(u_idx_hbm, ue_hbm, up_hbm, u_out_hbm, w)

    @pl.when(w >= SIDE_WORKERS)
    def _():
        side_worker(i_idx_hbm, ie_hbm, ip_hbm, i_out_hbm, w - SIDE_WORKERS)


def kernel(user_indices, item_indices, user_embedding_table,
           item_embedding_table, user_profiles, item_profiles):
    u_idx = user_indices.astype(jnp.int32)
    i_idx = item_indices.astype(jnp.int32)

    mesh = plsc.VectorSubcoreMesh(core_axis_name="c", subcore_axis_name="s")
    out_t = jax.ShapeDtypeStruct((EMBED_DIM, BATCH), jnp.float32)
    blk = pltpu.VMEM((NSLOT, EMBED_DIM, BLK), jnp.float32)

    run = pl.kernel(
        _embed_kernel,
        out_type=(out_t, out_t),
        mesh=mesh,
        compiler_params=pltpu.CompilerParams(needs_layout_passes=False),
        scratch_types=[
            pltpu.VMEM((PER_WORKER,), jnp.int32),
            blk, blk,
            pltpu.VMEM((EMBED_DIM, PER_WORKER), jnp.float32),
            pltpu.SemaphoreType.DMA,
            pltpu.SemaphoreType.DMA,
        ],
    )
    u_out_t, i_out_t = run(
        u_idx, i_idx,
        user_embedding_table.T, item_embedding_table.T,
        user_profiles.T, item_profiles.T)
    return (u_out_t.T, i_out_t.T)


# speedup vs baseline: 2.4151x; 1.0124x over previous
import jax
import jax.numpy as jnp
from jax import lax
from jax.experimental import pallas as pl
from jax.experimental.pallas import tpu as pltpu
from jax.experimental.pallas import tpu_sc as plsc

BATCH = 16384
EMBED_DIM = 32
NUM_CORES = 2
NUM_SUBCORES = 16
NUM_WORKERS = NUM_CORES * NUM_SUBCORES
SIDE_WORKERS = NUM_WORKERS // 2
PER_WORKER = BATCH // SIDE_WORKERS
BLK = 128
LANES = 16
GROUP = 16
NGROUPS = PER_WORKER // GROUP
NSLOT = 8


def _embed_kernel(
    u_idx_hbm, i_idx_hbm,
    ue_hbm, ie_hbm, up_hbm, ip_hbm,
    u_out_hbm, i_out_hbm,
    idx_v, e_blk, p_blk, out_v,
    sem_e, sem_p,
):
    w = lax.axis_index("s") * NUM_CORES + lax.axis_index("c")

    rows_lo = lax.iota(jnp.int32, LANES)
    rows_hi = rows_lo + LANES
    third = jnp.float32(1.0 / 3.0)

    def side_worker(idx_hbm, e_hbm, p_hbm, out_hbm, lw):
        b0 = lw * PER_WORKER
        pltpu.sync_copy(idx_hbm.at[pl.ds(b0, PER_WORKER)], idx_v)

        def fire(u, slot):
            ub = (u >> 7) * BLK
            pltpu.async_copy(e_hbm.at[:, pl.ds(ub, BLK)], e_blk.at[slot], sem_e)
            pltpu.async_copy(p_hbm.at[:, pl.ds(ub, BLK)], p_blk.at[slot], sem_p)

        def drain(slot):
            pltpu.make_async_copy(e_hbm.at[:, pl.ds(0, BLK)], e_blk.at[slot], sem_e).wait()
            pltpu.make_async_copy(p_hbm.at[:, pl.ds(0, BLK)], p_blk.at[slot], sem_p).wait()

        def process(u, slot, col):
            lane_vec = jnp.full((LANES,), u & (BLK - 1), jnp.int32)
            col_v = jnp.full((LANES,), col, jnp.int32)
            e_lo = plsc.load_gather(e_blk.at[slot], [rows_lo, lane_vec])
            e_hi = plsc.load_gather(e_blk.at[slot], [rows_hi, lane_vec])
            p_lo = plsc.load_gather(p_blk.at[slot], [rows_lo, lane_vec])
            p_hi = plsc.load_gather(p_blk.at[slot], [rows_hi, lane_vec])
            o_lo = (e_lo + p_lo + e_lo * p_lo) * third
            o_hi = (e_hi + p_hi + e_hi * p_hi) * third
            plsc.store_scatter(out_v, [rows_lo, col_v], o_lo)
            plsc.store_scatter(out_v, [rows_hi, col_v], o_hi)

        vec0 = idx_v[pl.ds(0, GROUP)]
        for k in range(NSLOT - 1):
            fire(vec0[k], k)

        @pl.loop(0, NGROUPS)
        def _(g):
            vec = idx_v[pl.ds(g * GROUP, GROUP)]
            nxt = idx_v[pl.ds((g + 1) * GROUP % PER_WORKER, GROUP)]
            for k in range(GROUP):
                ka = k + NSLOT - 1
                if ka < GROUP:
                    fire(vec[ka], ka & (NSLOT - 1))
                else:
                    fire(nxt[ka - GROUP], ka & (NSLOT - 1))
                drain(k & (NSLOT - 1))
                process(vec[k], k & (NSLOT - 1), g * GROUP + k)

        for k in range(NSLOT - 1):
            drain(k)

        pltpu.sync_copy(out_v, out_hbm.at[:, pl.ds(b0, PER_WORKER)])

    @pl.when(w < SIDE_WORKERS)
    def _():
        side_worker(u_idx_hbm, ue_hbm, up_hbm, u_out_hbm, w)

    @pl.when(w >= SIDE_WORKERS)
    def _():
        side_worker(i_idx_hbm, ie_hbm, ip_hbm, i_out_hbm, w - SIDE_WORKERS)


def kernel(user_indices, item_indices, user_embedding_table,
           item_embedding_table, user_profiles, item_profiles):
    u_idx = user_indices.astype(jnp.int32)
    i_idx = item_indices.astype(jnp.int32)

    mesh = plsc.VectorSubcoreMesh(core_axis_name="c", subcore_axis_name="s")
    out_t = jax.ShapeDtypeStruct((EMBED_DIM, BATCH), jnp.float32)
    blk = pltpu.VMEM((NSLOT, EMBED_DIM, BLK), jnp.float32)

    run = pl.kernel(
        _embed_kernel,
        out_type=(out_t, out_t),
        mesh=mesh,
        compiler_params=pltpu.CompilerParams(needs_layout_passes=False),
        scratch_types=[
            pltpu.VMEM((PER_WORKER,), jnp.int32),
            blk, blk,
            pltpu.VMEM((EMBED_DIM, PER_WORKER), jnp.float32),
            pltpu.SemaphoreType.DMA,
            pltpu.SemaphoreType.DMA,
        ],
    )
    u_out_t, i_out_t = run(
        u_idx, i_idx,
        user_embedding_table.T, item_embedding_table.T,
        user_profiles.T, item_profiles.T)
    return (u_out_t.T, i_out_t.T)
